# baseline (speedup 1.0000x reference)
import functools

import jax
import jax.numpy as jnp
from jax import lax
from jax.experimental import pallas as pl
from jax.experimental.pallas import tpu as pltpu
from jax.experimental.pallas import tpu_sc as plsc

NUM_ROLES = 6
D = 128
ROWS = 16384
COLS = 200
B = ROWS * COLS

NC = 2
NS = 16
NW = NC * NS
B_PER_W = B // NW

CHUNK = 128
N_CHUNKS = B_PER_W // CHUNK


@functools.partial(
    pl.kernel,
    mesh=plsc.VectorSubcoreMesh(core_axis_name="c", subcore_axis_name="s"),
    out_type=jax.ShapeDtypeStruct((B, D), jnp.float32),
    scratch_types=[
        pltpu.VMEM((CHUNK,), jnp.int32),
        pltpu.VMEM((CHUNK, D), jnp.float32),
        pltpu.SemaphoreType.DMA,
    ],
)
def _gather_rows(idx_hbm, table_hbm, out_hbm, idx_v, rows_v, sem):
    wid = lax.axis_index("s") * NC + lax.axis_index("c")
    base = wid * B_PER_W

    def step(g, carry):
        off = base + g * CHUNK
        pltpu.sync_copy(idx_hbm.at[pl.ds(off, CHUNK)], idx_v)
        pltpu.async_copy(table_hbm.at[idx_v], rows_v, sem).wait()
        pltpu.sync_copy(rows_v, out_hbm.at[pl.ds(off, CHUNK)])
        return carry

    lax.fori_loop(0, N_CHUNKS, step, 0)


def kernel(role_indices, embedding_weight):
    flat_idx = role_indices.reshape(B).astype(jnp.int32)
    out = _gather_rows(flat_idx, embedding_weight)
    return out.reshape(ROWS, COLS, D)


# speedup vs baseline: 1.0025x; 1.0025x over previous
import functools

import jax
import jax.numpy as jnp
from jax import lax
from jax.experimental import pallas as pl
from jax.experimental.pallas import tpu as pltpu
from jax.experimental.pallas import tpu_sc as plsc

NUM_ROLES = 6
D = 128
ROWS = 16384
COLS = 200
B = ROWS * COLS

NC = 2
NS = 16
NW = NC * NS
B_PER_W = B // NW

BLK = 256
GCHUNK = 128
K = BLK // GCHUNK
N_BLK = B_PER_W // BLK


@functools.partial(
    pl.kernel,
    mesh=plsc.VectorSubcoreMesh(core_axis_name="c", subcore_axis_name="s"),
    out_type=jax.ShapeDtypeStruct((B, D), jnp.float32),
    scratch_types=[
        pltpu.VMEM((2, BLK), jnp.int32),
        pltpu.VMEM((2, BLK, D), jnp.float32),
        pltpu.SemaphoreType.DMA,
        pltpu.SemaphoreType.DMA,
    ],
)
def _gather_rows(idx_hbm, table_hbm, out_hbm, idx_v, rows_v, sem_g, sem_w):
    wid = lax.axis_index("s") * NC + lax.axis_index("c")
    base = wid * B_PER_W

    def fire_gathers(i, h):
        for k in range(K):
            sl = pl.ds(k * GCHUNK, GCHUNK)
            pltpu.async_copy(table_hbm.at[idx_v.at[h, sl]], rows_v.at[h, sl],
                             sem_g)

    def drain(sem, h):
        pltpu.make_async_copy(out_hbm.at[pl.ds(0, BLK)], rows_v.at[h],
                              sem).wait()

    pltpu.sync_copy(idx_hbm.at[pl.ds(base, BLK)], idx_v.at[0])
    fire_gathers(0, 0)
    pltpu.sync_copy(idx_hbm.at[pl.ds(base + BLK, BLK)], idx_v.at[1])

    def step(i, carry):
        h = lax.rem(i, 2)
        drain(sem_g, h)

        @pl.when(i > 0)
        def _():
            drain(sem_w, 1 - h)

        @pl.when(i < N_BLK - 1)
        def _():
            fire_gathers(i + 1, 1 - h)

        pltpu.async_copy(rows_v.at[h], out_hbm.at[pl.ds(base + i * BLK, BLK)],
                         sem_w)

        @pl.when(i + 2 < N_BLK)
        def _():
            pltpu.sync_copy(idx_hbm.at[pl.ds(base + (i + 2) * BLK, BLK)],
                            idx_v.at[h])

        return carry

    lax.fori_loop(0, N_BLK, step, 0)
    drain(sem_w, (N_BLK - 1) % 2)


def kernel(role_indices, embedding_weight):
    flat_idx = role_indices.reshape(B).astype(jnp.int32)
    out = _gather_rows(flat_idx, embedding_weight)
    return out.reshape(ROWS, COLS, D)


# speedup vs baseline: 31.4720x; 31.3940x over previous
import functools

import jax
import jax.numpy as jnp
from jax import lax
from jax.experimental import pallas as pl
from jax.experimental.pallas import tpu as pltpu
from jax.experimental.pallas import tpu_sc as plsc

NUM_ROLES = 6
D = 128
ROWS = 16384
COLS = 200
B = ROWS * COLS

NC = 2
NS = 16
NW = NC * NS
B_PER_W = B // NW

BLK = 256
GCHUNK = 128
K = BLK // GCHUNK
N_BLK = B_PER_W // BLK


@functools.partial(
    pl.kernel,
    mesh=plsc.VectorSubcoreMesh(core_axis_name="c", subcore_axis_name="s"),
    out_type=jax.ShapeDtypeStruct((B, D), jnp.float32),
    scratch_types=[
        pltpu.VMEM((2, BLK), jnp.int32),
        pltpu.VMEM((2, BLK, D), jnp.float32),
        pltpu.VMEM_SHARED((NUM_ROLES, D), jnp.float32),
        pltpu.SemaphoreType.DMA,
        pltpu.SemaphoreType.DMA,
    ],
)
def _gather_rows(idx_hbm, table_hbm, out_hbm, idx_v, rows_v, table_v,
                 sem_g, sem_w):
    wid = lax.axis_index("s") * NC + lax.axis_index("c")
    base = wid * B_PER_W
    @pl.when(lax.axis_index("s") == 0)
    def _():
        pltpu.sync_copy(table_hbm, table_v)

    plsc.subcore_barrier()

    def fire_gathers(i, h):
        for k in range(K):
            sl = pl.ds(k * GCHUNK, GCHUNK)
            pltpu.async_copy(table_v.at[idx_v.at[h, sl]], rows_v.at[h, sl],
                             sem_g)

    def drain(sem, h):
        pltpu.make_async_copy(out_hbm.at[pl.ds(0, BLK)], rows_v.at[h],
                              sem).wait()

    pltpu.sync_copy(idx_hbm.at[pl.ds(base, BLK)], idx_v.at[0])
    fire_gathers(0, 0)
    pltpu.sync_copy(idx_hbm.at[pl.ds(base + BLK, BLK)], idx_v.at[1])

    def step(i, carry):
        h = lax.rem(i, 2)
        drain(sem_g, h)

        @pl.when(i > 0)
        def _():
            drain(sem_w, 1 - h)

        @pl.when(i < N_BLK - 1)
        def _():
            fire_gathers(i + 1, 1 - h)

        pltpu.async_copy(rows_v.at[h], out_hbm.at[pl.ds(base + i * BLK, BLK)],
                         sem_w)

        @pl.when(i + 2 < N_BLK)
        def _():
            pltpu.sync_copy(idx_hbm.at[pl.ds(base + (i + 2) * BLK, BLK)],
                            idx_v.at[h])

        return carry

    lax.fori_loop(0, N_BLK, step, 0)
    drain(sem_w, (N_BLK - 1) % 2)


def kernel(role_indices, embedding_weight):
    flat_idx = role_indices.reshape(B).astype(jnp.int32)
    out = _gather_rows(flat_idx, embedding_weight)
    return out.reshape(ROWS, COLS, D)
